# Initial kernel scaffold; baseline (speedup 1.0000x reference)
import jax
import jax.numpy as jnp
from jax.experimental import pallas as pl


def kernel(X, edge_weight, W_low, b_low, W_mid, b_mid, W_high, b_high, lowalpha, lowgamma, highalpha, highgamma, midalpha, midgamma, bias, edge_index):
    raise NotImplementedError("write your pallas kernel here")


# speedup vs baseline: 3.6352x; 3.6352x over previous
import functools

import jax
import jax.numpy as jnp
from jax import lax
from jax.experimental import pallas as pl
from jax.experimental.pallas import tpu as pltpu
from jax.experimental.pallas import tpu_sc as plsc

_NUM_CORES = 2
_NUM_SUBCORES = 16
_LANES = 16
_CHUNK = 128


def _make_spmm(n_pad, half, n_chunks):
  mesh = plsc.VectorSubcoreMesh(
      core_axis_name="c", subcore_axis_name="s",
      num_cores=_NUM_CORES, num_subcores=_NUM_SUBCORES)
  rows_per = n_pad // _NUM_SUBCORES

  @functools.partial(
      pl.kernel,
      out_type=jax.ShapeDtypeStruct((_NUM_CORES, n_pad, half), jnp.float32),
      mesh=mesh,
      scratch_types=[
          pltpu.VMEM((n_chunks, _CHUNK), jnp.int32),
          pltpu.VMEM((n_chunks, _CHUNK), jnp.int32),
          pltpu.VMEM((n_chunks, _CHUNK), jnp.float32),
          pltpu.VMEM((_CHUNK, half), jnp.float32),
          pltpu.VMEM_SHARED((n_pad, half), jnp.float32),
          pltpu.SemaphoreType.DMA,
      ],
  )
  def spmm(xs_hbm, src_hbm, dst_hbm, w_hbm, zeros_hbm, out_hbm,
           src_v, dst_v, w_v, rows_v, acc, sem):
    c = lax.axis_index("c")
    s = lax.axis_index("s")
    pltpu.sync_copy(src_hbm.at[c, s], src_v)
    pltpu.sync_copy(dst_hbm.at[s], dst_v)
    pltpu.sync_copy(w_hbm.at[s], w_v)
    r0 = s * rows_per
    pltpu.sync_copy(zeros_hbm.at[pl.ds(r0, rows_per)],
                    acc.at[pl.ds(r0, rows_per)])
    plsc.subcore_barrier()

    def chunk_body(j, carry):
      pltpu.async_copy(xs_hbm.at[src_v.at[j]], rows_v, sem).wait()

      def group_body(g, carry2):
        wv = w_v[j, pl.ds(g * _LANES, _LANES)]
        for r16 in range(_LANES):
          w = wv[r16]
          row = g * _LANES + r16
          for f in range(half // _LANES):
            sl = pl.ds(f * _LANES, _LANES)
            rows_v[row, sl] = rows_v[row, sl] * w
        return carry2

      lax.fori_loop(0, _CHUNK // _LANES, group_body, 0)
      pltpu.sync_copy(rows_v, acc.at[dst_v.at[j]], add=True)
      return carry

    lax.fori_loop(0, n_chunks, chunk_body, 0)
    plsc.subcore_barrier()
    pltpu.sync_copy(acc.at[pl.ds(r0, rows_per)],
                    out_hbm.at[c, pl.ds(r0, rows_per)])

  return spmm


def _combine_body(scal_ref, x_ref, lx_ref, wl_ref, wm_ref, wh_ref, b4_ref,
                  o_ref):
  la = jnp.clip(scal_ref[0], 0.0, 1.0)
  lg = jnp.maximum(scal_ref[1], 0.0)
  ha = jnp.clip(scal_ref[2], 0.0, 1.0)
  hg = jnp.maximum(scal_ref[3], 0.0)
  ma = jnp.clip(scal_ref[4], 0.0, 1.0)
  mg = jnp.maximum(scal_ref[5], 0.0)
  x = x_ref[...]
  lx = jnp.concatenate([lx_ref[0], lx_ref[1]], axis=-1)
  h = x - lx
  hsq = h * h
  wl = wl_ref[...]
  wm = wm_ref[...]
  wh = wh_ref[...]
  a_mat = (-la * lg) * wl + (ha * hg) * wh
  b_mat = lg * wl + ((1.0 - 2.0 * ha) * hg) * wh - (ma * mg) * wm
  c_mat = mg * wm
  acc = jnp.dot(h, a_mat, preferred_element_type=jnp.float32)
  acc = acc + jnp.dot(x, b_mat, preferred_element_type=jnp.float32)
  acc = acc + jnp.dot(hsq, c_mat, preferred_element_type=jnp.float32)
  bsum = jnp.sum(b4_ref[...], axis=0)
  o_ref[...] = acc + bsum[None, :]


def _make_combine(n_nodes, d, half, block_rows):
  grid = n_nodes // block_rows
  return pl.pallas_call(
      _combine_body,
      grid=(grid,),
      in_specs=[
          pl.BlockSpec(memory_space=pltpu.SMEM),
          pl.BlockSpec((block_rows, d), lambda i: (i, 0)),
          pl.BlockSpec((2, block_rows, half), lambda i: (0, i, 0)),
          pl.BlockSpec((d, d), lambda i: (0, 0)),
          pl.BlockSpec((d, d), lambda i: (0, 0)),
          pl.BlockSpec((d, d), lambda i: (0, 0)),
          pl.BlockSpec((4, d), lambda i: (0, 0)),
      ],
      out_specs=pl.BlockSpec((block_rows, d), lambda i: (i, 0)),
      out_shape=jax.ShapeDtypeStruct((n_nodes, d), jnp.float32),
  )


def kernel(X, edge_weight, W_low, b_low, W_mid, b_mid, W_high, b_high,
           lowalpha, lowgamma, highalpha, highgamma, midalpha, midgamma,
           bias, edge_index):
  n_nodes, d = X.shape
  e = edge_weight.shape[0]
  half = d // 2
  edges_per_tile = _NUM_SUBCORES * _CHUNK
  n_chunks = -(-e // edges_per_tile)
  ep = n_chunks * edges_per_tile
  pad = ep - e

  src = edge_index[0]
  dst = edge_index[1]
  if pad:
    zi = jnp.zeros((pad,), jnp.int32)
    src = jnp.concatenate([src, zi])
    dst = jnp.concatenate([dst, zi])
    edge_weight = jnp.concatenate(
        [edge_weight, jnp.zeros((pad,), jnp.float32)])

  xs = jnp.concatenate([X[:, :half], X[:, half:]], axis=0)
  src2 = jnp.stack([src, src + n_nodes]).reshape(
      _NUM_CORES, _NUM_SUBCORES, n_chunks, _CHUNK)
  dst3 = dst.reshape(_NUM_SUBCORES, n_chunks, _CHUNK)
  w3 = edge_weight.reshape(_NUM_SUBCORES, n_chunks, _CHUNK)
  n_pad = -(-n_nodes // (8 * _NUM_SUBCORES)) * (8 * _NUM_SUBCORES)
  zeros = jnp.zeros((n_pad, half), jnp.float32)

  lx2 = _make_spmm(n_pad, half, n_chunks)(xs, src2, dst3, w3, zeros)

  scal = jnp.stack([lowalpha[0], lowgamma[0], highalpha[0], highgamma[0],
                    midalpha[0], midgamma[0]])
  b4 = jnp.stack([b_low, b_mid, b_high, bias])
  return _make_combine(n_nodes, d, half, 1000)(
      scal, X, lx2, W_low, W_mid, W_high, b4)


# speedup vs baseline: 5.2916x; 1.4557x over previous
import functools

import jax
import jax.numpy as jnp
from jax import lax
from jax.experimental import pallas as pl
from jax.experimental.pallas import tpu as pltpu
from jax.experimental.pallas import tpu_sc as plsc

_NUM_CORES = 2
_NUM_SUBCORES = 16
_LANES = 16
_CHUNK = 112
_UNROLL = 6
_NBUF = 3


def _make_spmm(n_pad, half, n_chunks):
  assert n_chunks % _UNROLL == 0
  mesh = plsc.VectorSubcoreMesh(
      core_axis_name="c", subcore_axis_name="s",
      num_cores=_NUM_CORES, num_subcores=_NUM_SUBCORES)
  rows_per = n_pad // _NUM_SUBCORES

  @functools.partial(
      pl.kernel,
      out_type=jax.ShapeDtypeStruct((_NUM_CORES, n_pad, half), jnp.float32),
      mesh=mesh,
      scratch_types=[
          [pltpu.VMEM((_CHUNK, half), jnp.float32) for _ in range(_NBUF)],
          [pltpu.VMEM((2, _CHUNK), jnp.int32) for _ in range(_UNROLL)],
          [pltpu.VMEM((_CHUNK,), jnp.float32) for _ in range(_UNROLL)],
          pltpu.VMEM_SHARED((n_pad, half), jnp.float32),
          [pltpu.SemaphoreType.DMA for _ in range(_NBUF)],
          [pltpu.SemaphoreType.DMA for _ in range(_NBUF)],
          [pltpu.SemaphoreType.DMA for _ in range(_UNROLL)],
          [pltpu.SemaphoreType.DMA for _ in range(_UNROLL)],
      ],
  )
  def spmm(xs_hbm, meta_hbm, w_hbm, zeros_hbm, out_hbm,
           bufs, metas, wbufs, acc, gsems, ssems, msems, wsems):
    c = lax.axis_index("c")
    s = lax.axis_index("s")
    r0 = s * rows_per
    pltpu.sync_copy(zeros_hbm.at[pl.ds(r0, rows_per)],
                    acc.at[pl.ds(r0, rows_per)])

    def meta_fetch(j, mb):
      return pltpu.make_async_copy(meta_hbm.at[c, s, j], metas[mb], msems[mb])

    def w_fetch(j, mb):
      return pltpu.make_async_copy(w_hbm.at[s, j], wbufs[mb], wsems[mb])

    def gather(k):
      b = k % _NBUF
      return pltpu.make_async_copy(
          xs_hbm.at[metas[k % _UNROLL].at[0]], bufs[b], gsems[b])

    def scatter(k):
      b = k % _NBUF
      return pltpu.make_async_copy(
          bufs[b], acc.at[metas[k % _UNROLL].at[1]], ssems[b])

    plsc.subcore_barrier()

    for j in range(_NBUF):
      meta_fetch(j, j).start()
      w_fetch(j, j).start()
    meta_fetch(0, 0).wait()
    gather(0).start()
    meta_fetch(1, 1).wait()
    gather(1).start()

    def pair_body(i, carry):
      for k in range(_UNROLL):
        jj = i * _UNROLL + k
        gather(k).wait()
        w_fetch(0, k).wait()

        b = k % _NBUF

        def group_body(g, carry2, b=b, k=k):
          wv = wbufs[k][pl.ds(g * _LANES, _LANES)]
          for r16 in range(_LANES):
            w = wv[r16]
            row = g * _LANES + r16
            for f in range(half // _LANES):
              sl = pl.ds(f * _LANES, _LANES)
              bufs[b][row, sl] = bufs[b][row, sl] * w
          return carry2

        lax.fori_loop(0, _CHUNK // _LANES, group_body, 0)

        if k == 0:
          @pl.when(jj >= 1)
          def _w0():
            scatter(k - 1).wait()
        else:
          scatter(k - 1).wait()

        @pl.when(jj + _NBUF < n_chunks)
        def _m(jj=jj, k=k):
          meta_fetch(jj + _NBUF, (k + _NBUF) % _UNROLL).start()
          w_fetch(jj + _NBUF, (k + _NBUF) % _UNROLL).start()

        @pl.when(jj + 2 < n_chunks)
        def _g(k=k):
          meta_fetch(0, (k + 2) % _UNROLL).wait()
          gather(k + 2).start()


        scatter(k).start(add=True)
      return carry

    lax.fori_loop(0, n_chunks // _UNROLL, pair_body, 0)
    scatter(n_chunks - 1).wait()
    plsc.subcore_barrier()
    pltpu.sync_copy(acc.at[pl.ds(r0, rows_per)],
                    out_hbm.at[c, pl.ds(r0, rows_per)])

  return spmm


def _combine_body(scal_ref, x_ref, lx_ref, wl_ref, wm_ref, wh_ref, b4_ref,
                  o_ref):
  la = jnp.clip(scal_ref[0], 0.0, 1.0)
  lg = jnp.maximum(scal_ref[1], 0.0)
  ha = jnp.clip(scal_ref[2], 0.0, 1.0)
  hg = jnp.maximum(scal_ref[3], 0.0)
  ma = jnp.clip(scal_ref[4], 0.0, 1.0)
  mg = jnp.maximum(scal_ref[5], 0.0)
  x = x_ref[...]
  lx = jnp.concatenate([lx_ref[0], lx_ref[1]], axis=-1)
  h = x - lx
  hsq = h * h
  wl = wl_ref[...]
  wm = wm_ref[...]
  wh = wh_ref[...]
  a_mat = (-la * lg) * wl + (ha * hg) * wh
  b_mat = lg * wl + ((1.0 - 2.0 * ha) * hg) * wh - (ma * mg) * wm
  c_mat = mg * wm
  acc = jnp.dot(h, a_mat, preferred_element_type=jnp.float32)
  acc = acc + jnp.dot(x, b_mat, preferred_element_type=jnp.float32)
  acc = acc + jnp.dot(hsq, c_mat, preferred_element_type=jnp.float32)
  bsum = jnp.sum(b4_ref[...], axis=0)
  o_ref[...] = acc + bsum[None, :]


def _make_combine(n_nodes, d, half, block_rows):
  grid = n_nodes // block_rows
  return pl.pallas_call(
      _combine_body,
      grid=(grid,),
      in_specs=[
          pl.BlockSpec(memory_space=pltpu.SMEM),
          pl.BlockSpec((block_rows, d), lambda i: (i, 0)),
          pl.BlockSpec((2, block_rows, half), lambda i: (0, i, 0)),
          pl.BlockSpec((d, d), lambda i: (0, 0)),
          pl.BlockSpec((d, d), lambda i: (0, 0)),
          pl.BlockSpec((d, d), lambda i: (0, 0)),
          pl.BlockSpec((4, d), lambda i: (0, 0)),
      ],
      out_specs=pl.BlockSpec((block_rows, d), lambda i: (i, 0)),
      out_shape=jax.ShapeDtypeStruct((n_nodes, d), jnp.float32),
  )


def kernel(X, edge_weight, W_low, b_low, W_mid, b_mid, W_high, b_high,
           lowalpha, lowgamma, highalpha, highgamma, midalpha, midgamma,
           bias, edge_index):
  n_nodes, d = X.shape
  e = edge_weight.shape[0]
  half = d // 2
  edges_per_tile = _NUM_SUBCORES * _CHUNK
  n_chunks = -(-e // edges_per_tile)
  n_chunks = -(-n_chunks // _UNROLL) * _UNROLL
  ep = n_chunks * edges_per_tile
  pad = ep - e

  src = edge_index[0]
  dst = edge_index[1]
  if pad:
    zi = jnp.zeros((pad,), jnp.int32)
    src = jnp.concatenate([src, zi])
    dst = jnp.concatenate([dst, zi])
    edge_weight = jnp.concatenate(
        [edge_weight, jnp.zeros((pad,), jnp.float32)])

  xs = jnp.concatenate([X[:, :half], X[:, half:]], axis=0)
  src_t = src.reshape(_NUM_SUBCORES, n_chunks, _CHUNK)
  dst_t = dst.reshape(_NUM_SUBCORES, n_chunks, _CHUNK)
  meta = jnp.stack([
      jnp.stack([src_t, dst_t], axis=2),
      jnp.stack([src_t + n_nodes, dst_t], axis=2),
  ])
  w_t = edge_weight.reshape(_NUM_SUBCORES, n_chunks, _CHUNK)

  n_pad = -(-n_nodes // (8 * _NUM_SUBCORES)) * (8 * _NUM_SUBCORES)
  zeros = jnp.zeros((n_pad, half), jnp.float32)

  lx2 = _make_spmm(n_pad, half, n_chunks)(xs, meta, w_t, zeros)

  scal = jnp.stack([lowalpha[0], lowgamma[0], highalpha[0], highgamma[0],
                    midalpha[0], midgamma[0]])
  b4 = jnp.stack([b_low, b_mid, b_high, bias])
  return _make_combine(n_nodes, d, half, 1000)(
      scal, X, lx2, W_low, W_mid, W_high, b4)
